# baseline (speedup 1.0000x reference)
import functools

import jax
import jax.numpy as jnp
from jax import lax
from jax.experimental import pallas as pl
from jax.experimental.pallas import tpu as pltpu
from jax.experimental.pallas import tpu_sc as plsc

_NC = 2
_NS = 16
_NW = _NC * _NS

_LPB = 40
_RPB = 3 * _LPB
_NBUF = 4
_GRP = 32
_GIDX = _GRP * _RPB


@functools.partial(jax.jit, static_argnums=(2, 3))
def _gather_flat(idx, table_pad, bb, ll):
    n_tok = bb * ll
    per_w_tok = n_tok // _NW
    blocks_per_b = ll // _LPB
    n_groups = per_w_tok // (_GRP * _LPB)
    n_inner = _GRP // _NBUF
    mesh = plsc.VectorSubcoreMesh(core_axis_name="c", subcore_axis_name="s")

    @functools.partial(
        pl.kernel,
        out_type=jax.ShapeDtypeStruct((bb, ll, 192), jnp.float32),
        mesh=mesh,
        scratch_types=[
            pltpu.VMEM((_GIDX,), jnp.int32),
            pltpu.VMEM((_NBUF, _RPB, 128), jnp.float32),
            pltpu.VMEM((_NBUF, _LPB, 192), jnp.float32),
            pltpu.SemaphoreType.DMA,
            pltpu.SemaphoreType.DMA,
            pltpu.SemaphoreType.DMA,
            pltpu.SemaphoreType.DMA,
            pltpu.SemaphoreType.DMA,
        ],
    )
    def emb(idx_hbm, table_hbm, out_hbm, idx_v, gbuf, pret, gsem, w0, w1, w2, w3):
        wid = lax.axis_index("s") * _NC + lax.axis_index("c")
        tok0 = wid * per_w_tok
        blk0 = wid * (per_w_tok // _LPB)
        wsems = (w0, w1, w2, w3)

        def group(g, carry):
            gtok0 = tok0 + g * (_GRP * _LPB)
            pltpu.sync_copy(
                idx_hbm.at[pl.ds(pl.multiple_of(gtok0 * 3, _GIDX), _GIDX)],
                idx_v,
            )

            def inner(i2, carry2):
                first = (g == 0) & (i2 == 0)
                cps = []
                for p in range(_NBUF):
                    beta = i2 * _NBUF + p
                    cps.append(
                        pltpu.async_copy(
                            table_hbm.at[idx_v.at[pl.ds(beta * _RPB, _RPB)]],
                            gbuf.at[p],
                            gsem,
                        )
                    )
                for p in range(_NBUF):
                    beta = i2 * _NBUF + p
                    blk = blk0 + g * _GRP + beta
                    bidx = blk // blocks_per_b
                    l0 = pl.multiple_of((blk % blocks_per_b) * _LPB, _LPB)
                    out_slice = out_hbm.at[bidx, pl.ds(l0, _LPB)]
                    cps[p].wait()
                    @pl.when(jnp.logical_not(first))
                    def _():
                        pltpu.make_async_copy(
                            pret.at[p], out_slice, wsems[p]
                        ).wait()
                    gp = gbuf.at[p]
                    pp = pret.at[p]
                    for r in range(_RPB):
                        dl = r // 3
                        dc = (r % 3) * 64
                        for k in range(4):
                            pp[dl, pl.ds(dc + k * 16, 16)] = gp[
                                r, pl.ds(k * 16, 16)
                            ]
                    pltpu.async_copy(pret.at[p], out_slice, wsems[p])
                return carry2

            lax.fori_loop(0, n_inner, inner, 0)
            return carry

        lax.fori_loop(0, n_groups, group, 0)
        for p in range(_NBUF):
            blk = blk0 + n_groups * _GRP - _NBUF + p
            bidx = blk // blocks_per_b
            l0 = pl.multiple_of((blk % blocks_per_b) * _LPB, _LPB)
            pltpu.make_async_copy(
                pret.at[p],
                out_hbm.at[bidx, pl.ds(l0, _LPB)],
                wsems[p],
            ).wait()

    return emb(idx, table_pad)


def kernel(phoneme_tensor, embedding_weight):
    b, l, c = phoneme_tensor.shape
    v, d = embedding_weight.shape
    n = b * l * c
    assert c == 3 and d == 64
    per_w_tok = (n // 3) // _NW
    assert per_w_tok % (_GRP * _LPB) == 0 and l % _LPB == 0
    idx = phoneme_tensor.astype(jnp.int32).reshape(n)
    table_pad = jnp.pad(embedding_weight, ((0, 0), (0, 128 - d)))
    return _gather_flat(idx, table_pad, b, l)
